# baseline (speedup 1.0000x reference)
import functools

import jax
import jax.numpy as jnp
from jax import lax
from jax.experimental import pallas as pl
from jax.experimental.pallas import tpu as pltpu
from jax.experimental.pallas import tpu_sc as plsc

N = 8 * 512 * 512
NBF = 65536
NBC = 4096
NC, NS, L = 2, 16, 16
NW = NC * NS
PER_W = N // NW
C = 4096
CHUNKS = PER_W // C
VPC = C // L
FSTRIPE = 2 * NBF // NS
CSTRIPE = 2 * NBC // NS
SLAB = L * NBC


def _hist_body(lg_hbm, tg_hbm, out_cnt, out_asum,
               lgb0, lgb1, tgb0, tgb1, av0, av1, ai0, ai1, fold, slab, tasum,
               semlg0, semlg1, semtg0, semtg1, sems0, sems1):
    c = lax.axis_index("c")
    s = lax.axis_index("s")
    wid = c * NS + s
    base = wid * PER_W
    lgb = (lgb0, lgb1)
    tgb = (tgb0, tgb1)
    av = (av0, av1)
    ai = (ai0, ai1)
    semlg = (semlg0, semlg1)
    semtg = (semtg0, semtg1)
    sems = (sems0, sems1)
    lanes = lax.broadcasted_iota(jnp.int32, (L,), 0)

    def zslab(j, _):
        slab[pl.ds(j * L, L)] = jnp.zeros((L,), jnp.int32)
        return 0
    lax.fori_loop(0, SLAB // L, zslab, 0)

    def zav(j, _):
        av0[pl.ds(j * L, L)] = jnp.zeros((L,), jnp.float32)
        return 0
    lax.fori_loop(0, VPC, zav, 0)
    pltpu.sync_copy(av0, tasum.at[pl.ds(s * FSTRIPE, C)])
    pltpu.sync_copy(av0, tasum.at[pl.ds(s * FSTRIPE + C, C)])
    plsc.subcore_barrier()

    pltpu.async_copy(lg_hbm.at[pl.ds(base, C)], lgb0, semlg0)
    pltpu.async_copy(tg_hbm.at[pl.ds(base, C)], tgb0, semtg0)

    def chunk_pair(g, _):
        for b in range(2):
            k = 2 * g + b
            nxt = k + 1

            @pl.when(nxt < CHUNKS)
            def _():
                noff = base + nxt * C
                pltpu.async_copy(lg_hbm.at[pl.ds(noff, C)], lgb[1 - b],
                                 semlg[1 - b])
                pltpu.async_copy(tg_hbm.at[pl.ds(noff, C)], tgb[1 - b],
                                 semtg[1 - b])

            off = base + k * C
            pltpu.make_async_copy(lg_hbm.at[pl.ds(off, C)], lgb[b],
                                  semlg[b]).wait()
            pltpu.make_async_copy(tg_hbm.at[pl.ds(off, C)], tgb[b],
                                  semtg[b]).wait()

            @pl.when(k >= 2)
            def _():
                pltpu.make_async_copy(av[b], tasum.at[ai[b]], sems[b]).wait()

            def vec_body(j, _, _b=b):
                sl = pl.ds(j * L, L)
                x = lgb[_b][sl]
                l = tgb[_b][sl]
                lf = l.astype(jnp.float32)
                e = 1.0 - x * (2.0 * lf - 1.0)
                a = jnp.maximum(e, 0.0)
                bu = lax.bitcast_convert_type(e, jnp.uint32)
                negm = lax.bitcast_convert_type(e, jnp.int32) < 0
                u = jnp.where(negm, ~bu, bu | jnp.uint32(0x80000000))
                inv = ~u
                b16 = (inv >> 16).astype(jnp.int32)
                b12 = (inv >> 20).astype(jnp.int32)
                plsc.addupdate_scatter(slab, [lanes * NBC + b12],
                                       1 + (l << 16))
                av[_b][sl] = a
                ai[_b][sl] = b16 + (l << 16)
                return 0
            lax.fori_loop(0, VPC, vec_body, 0)
            pltpu.async_copy(av[b], tasum.at[ai[b]], sems[b], add=True)
        return 0
    lax.fori_loop(0, CHUNKS // 2, chunk_pair, 0)
    pltpu.make_async_copy(av0, tasum.at[ai0], sems0).wait()
    pltpu.make_async_copy(av1, tasum.at[ai1], sems1).wait()
    plsc.subcore_barrier()

    def fold_body(i, _):
        sl = pl.ds(i * L, L)
        acc = slab[sl]
        for t in range(1, L):
            acc = acc + slab[pl.ds(t * NBC + i * L, L)]
        fold[sl] = acc
        return 0
    lax.fori_loop(0, NBC // L, fold_body, 0)
    pltpu.sync_copy(fold, out_cnt.at[wid])

    pltpu.sync_copy(tasum.at[pl.ds(s * FSTRIPE, C)], lgb0)
    pltpu.sync_copy(tasum.at[pl.ds(s * FSTRIPE + C, C)], lgb1)
    for i in range(CSTRIPE // L):
        buf = lgb0 if i < CSTRIPE // L // 2 else lgb1
        lo = i * 16 * L - (C if i >= CSTRIPE // L // 2 else 0)
        acc = jnp.zeros((L,), jnp.float32)
        for f in range(16):
            acc = acc + plsc.load_gather(buf, [lo + lanes * 16 + f])
        av0[pl.ds(i * L, L)] = acc
    pltpu.sync_copy(av0.at[pl.ds(0, CSTRIPE)],
                    out_asum.at[c, pl.ds(s * CSTRIPE, CSTRIPE)])


def _scan_body(cnt_hbm, asum_hbm, out_hbm,
               cbuf, nbuf, pbuf, ambuf, apbuf, tbuf, stage, stagef, exv, exvf,
               outv, exch, exch2):
    c = lax.axis_index("c")
    s = lax.axis_index("s")
    STR = NBC // NS
    b0 = s * STR

    for r in range(NW):
        pltpu.sync_copy(cnt_hbm.at[r, pl.ds(b0, STR)],
                        cbuf.at[pl.ds(r * STR, STR)])
    pltpu.sync_copy(asum_hbm.at[0, pl.ds(b0, STR)], ambuf)
    pltpu.sync_copy(asum_hbm.at[1, pl.ds(b0, STR)], tbuf)
    pltpu.sync_copy(asum_hbm.at[0, pl.ds(NBC + b0, STR)], apbuf)

    def addam(j, _):
        sl = pl.ds(j * L, L)
        ambuf[sl] = ambuf[sl] + tbuf[sl]
        return 0
    lax.fori_loop(0, STR // L, addam, 0)
    pltpu.sync_copy(asum_hbm.at[1, pl.ds(NBC + b0, STR)], tbuf)

    def addap(j, _):
        sl = pl.ds(j * L, L)
        apbuf[sl] = apbuf[sl] + tbuf[sl]
        return 0
    lax.fori_loop(0, STR // L, addap, 0)

    def merge_body(j, carry):
        sn, sp = carry
        sl = pl.ds(j * L, L)
        tot = jnp.zeros((L,), jnp.int32)
        pos = jnp.zeros((L,), jnp.int32)
        for r in range(NW):
            v = cbuf[pl.ds(r * STR + j * L, L)]
            tot = tot + (v & 0xFFFF)
            pos = pos + lax.shift_right_logical(v, 16)
        neg = tot - pos
        nbuf[sl] = neg
        pbuf[sl] = pos
        return sn + jnp.sum(neg), sp + jnp.sum(pos)
    sneg, spos = lax.fori_loop(0, STR // L, merge_body,
                               (jnp.int32(0), jnp.int32(0)))

    lanes = lax.broadcasted_iota(jnp.int32, (L,), 0)
    stage[...] = jnp.where(lanes == 0, sneg, jnp.where(lanes == 1, spos, 0))
    pltpu.sync_copy(stage, exch.at[pl.ds(s * L, L)])
    plsc.subcore_barrier()
    pltpu.sync_copy(exch, exv)
    negs_all = plsc.load_gather(exv, [lanes * L])
    poss_all = plsc.load_gather(exv, [lanes * L + 1])
    qbase = jnp.sum(jnp.where(lanes < s, negs_all, 0))
    rbase = jnp.sum(jnp.where(lanes < s, poss_all, 0))
    pf = jnp.sum(poss_all).astype(jnp.float32)

    def scan_body(j, carry):
        qc, rc, acc = carry
        sl = pl.ds(j * L, L)
        neg = nbuf[sl]
        pos = pbuf[sl]
        qv = plsc.cumsum(neg) - neg + qc
        rv = plsc.cumsum(pos) - pos + rc
        qf = qv.astype(jnp.float32)
        rf = rv.astype(jnp.float32)
        negf = neg.astype(jnp.float32)
        posf = pos.astype(jnp.float32)
        am = ambuf[sl]
        ap = apbuf[sl]
        gplus = 1.0 / jnp.maximum(pf + qf + 0.5 * negf, 0.25)
        u0 = pf + qf + 0.5 * (negf - 1.0)
        gminus = (pf - rf - 0.5 * posf) / jnp.maximum(u0 * (u0 + 1.0), 0.25)
        acc = acc + ap * gplus + am * gminus
        return qc + jnp.sum(neg), rc + jnp.sum(pos), acc

    _, _, acc = lax.fori_loop(0, STR // L, scan_body,
                              (qbase, rbase, jnp.zeros((L,), jnp.float32)))
    part = jnp.sum(acc)
    stagef[...] = jnp.where(lanes == 0, part, 0.0)
    pltpu.sync_copy(stagef, exch2.at[pl.ds(s * L, L)])
    plsc.subcore_barrier()

    @pl.when(jnp.logical_and(c == 0, s == 0))
    def _():
        pltpu.sync_copy(exch2, exvf)
        parts = plsc.load_gather(exvf, [lanes * L])
        total = jnp.sum(parts)
        outv[...] = jnp.full((L,), total, jnp.float32)
        pltpu.sync_copy(outv, out_hbm)


@functools.partial(jax.jit, static_argnames=())
def kernel(logits, targets):
    lg = logits.reshape(-1)
    tg = targets.reshape(-1)
    mesh = plsc.VectorSubcoreMesh(core_axis_name="c", subcore_axis_name="s")
    params = pltpu.CompilerParams(needs_layout_passes=False)

    hist = pl.kernel(
        _hist_body,
        out_type=(
            jax.ShapeDtypeStruct((NW, NBC), jnp.int32),
            jax.ShapeDtypeStruct((NC, 2 * NBC), jnp.float32),
        ),
        mesh=mesh,
        scratch_types=[
            pltpu.VMEM((C,), jnp.float32),
            pltpu.VMEM((C,), jnp.float32),
            pltpu.VMEM((C,), jnp.int32),
            pltpu.VMEM((C,), jnp.int32),
            pltpu.VMEM((C,), jnp.float32),
            pltpu.VMEM((C,), jnp.float32),
            pltpu.VMEM((C,), jnp.int32),
            pltpu.VMEM((C,), jnp.int32),
            pltpu.VMEM((NBC,), jnp.int32),
            pltpu.VMEM((SLAB,), jnp.int32),
            pltpu.VMEM_SHARED((2 * NBF,), jnp.float32),
            pltpu.SemaphoreType.DMA,
            pltpu.SemaphoreType.DMA,
            pltpu.SemaphoreType.DMA,
            pltpu.SemaphoreType.DMA,
            pltpu.SemaphoreType.DMA,
            pltpu.SemaphoreType.DMA,
        ],
        compiler_params=params,
    )
    cnt, asum = hist(lg, tg)

    scan = pl.kernel(
        _scan_body,
        out_type=jax.ShapeDtypeStruct((L,), jnp.float32),
        mesh=plsc.VectorSubcoreMesh(core_axis_name="c", subcore_axis_name="s"),
        scratch_types=[
            pltpu.VMEM((NW * (NBC // NS),), jnp.int32),
            pltpu.VMEM((NBC // NS,), jnp.int32),
            pltpu.VMEM((NBC // NS,), jnp.int32),
            pltpu.VMEM((NBC // NS,), jnp.float32),
            pltpu.VMEM((NBC // NS,), jnp.float32),
            pltpu.VMEM((NBC // NS,), jnp.float32),
            pltpu.VMEM((L,), jnp.int32),
            pltpu.VMEM((L,), jnp.float32),
            pltpu.VMEM((NS * L,), jnp.int32),
            pltpu.VMEM((NS * L,), jnp.float32),
            pltpu.VMEM((L,), jnp.float32),
            pltpu.VMEM_SHARED((NS * L,), jnp.int32),
            pltpu.VMEM_SHARED((NS * L,), jnp.float32),
        ],
        compiler_params=params,
    )
    out = scan(cnt, asum)
    return out[0]


# speedup vs baseline: 2.0316x; 2.0316x over previous
import functools

import jax
import jax.numpy as jnp
from jax import lax
from jax.experimental import pallas as pl
from jax.experimental.pallas import tpu as pltpu
from jax.experimental.pallas import tpu_sc as plsc

N = 8 * 512 * 512
NBF = 65536
NBC = 4096
NC, NS, L = 2, 16, 16
NW = NC * NS
PER_W = N // NW
C = 4096
CHUNKS = PER_W // C
VPC = C // L
FSTRIPE = 2 * NBF // NS
CSTRIPE = 2 * NBC // NS
SLAB = L * NBC
STR = NBC // NS


def _hist_body(lg_hbm, tg_hbm, out_cnt, out_asum,
               lgb0, lgb1, tgb0, tgb1, av0, av1, ai0, ai1, fold, slab, tasum,
               semlg0, semlg1, semtg0, semtg1, sems0, sems1):
    c = lax.axis_index("c")
    s = lax.axis_index("s")
    wid = c * NS + s
    base = wid * PER_W
    lgb = (lgb0, lgb1)
    tgb = (tgb0, tgb1)
    av = (av0, av1)
    ai = (ai0, ai1)
    semlg = (semlg0, semlg1)
    semtg = (semtg0, semtg1)
    sems = (sems0, sems1)
    lanes = lax.broadcasted_iota(jnp.int32, (L,), 0)
    zi = jnp.zeros((L,), jnp.int32)
    zf = jnp.zeros((L,), jnp.float32)

    @plsc.parallel_loop(0, SLAB // L, 1, unroll=8)
    def _(j):
        slab[pl.ds(j * L, L)] = zi

    @plsc.parallel_loop(0, VPC, 1, unroll=8)
    def _(j):
        av0[pl.ds(j * L, L)] = zf

    pltpu.sync_copy(av0, tasum.at[pl.ds(s * FSTRIPE, C)])
    pltpu.sync_copy(av0, tasum.at[pl.ds(s * FSTRIPE + C, C)])
    plsc.subcore_barrier()

    pltpu.async_copy(lg_hbm.at[pl.ds(base, C)], lgb0, semlg0)
    pltpu.async_copy(tg_hbm.at[pl.ds(base, C)], tgb0, semtg0)

    def chunk_pair(g, _):
        for b in range(2):
            k = 2 * g + b
            nxt = k + 1

            @pl.when(nxt < CHUNKS)
            def _():
                noff = base + nxt * C
                pltpu.async_copy(lg_hbm.at[pl.ds(noff, C)], lgb[1 - b],
                                 semlg[1 - b])
                pltpu.async_copy(tg_hbm.at[pl.ds(noff, C)], tgb[1 - b],
                                 semtg[1 - b])

            off = base + k * C
            pltpu.make_async_copy(lg_hbm.at[pl.ds(off, C)], lgb[b],
                                  semlg[b]).wait()
            pltpu.make_async_copy(tg_hbm.at[pl.ds(off, C)], tgb[b],
                                  semtg[b]).wait()

            @pl.when(k >= 2)
            def _():
                pltpu.make_async_copy(av[b], tasum.at[ai[b]], sems[b]).wait()

            lgbb, tgbb, avb, aib = lgb[b], tgb[b], av[b], ai[b]

            @plsc.parallel_loop(0, VPC, 1, unroll=4)
            def _(j):
                sl = pl.ds(j * L, L)
                x = lgbb[sl]
                l = tgbb[sl]
                lf = l.astype(jnp.float32)
                e = 1.0 - x * (2.0 * lf - 1.0)
                a = jnp.maximum(e, 0.0)
                bu = lax.bitcast_convert_type(e, jnp.uint32)
                negm = lax.bitcast_convert_type(e, jnp.int32) < 0
                u = jnp.where(negm, ~bu, bu | jnp.uint32(0x80000000))
                inv = ~u
                b16 = (inv >> 16).astype(jnp.int32)
                b12 = (inv >> 20).astype(jnp.int32)
                plsc.addupdate_scatter(slab, [lanes * NBC + b12],
                                       1 + (l << 16))
                avb[sl] = a
                aib[sl] = b16 + (l << 16)

            pltpu.async_copy(av[b], tasum.at[ai[b]], sems[b], add=True)
        return 0
    lax.fori_loop(0, CHUNKS // 2, chunk_pair, 0)
    pltpu.make_async_copy(av0, tasum.at[ai0], sems0).wait()
    pltpu.make_async_copy(av1, tasum.at[ai1], sems1).wait()
    plsc.subcore_barrier()

    @plsc.parallel_loop(0, NBC // L, 1, unroll=2)
    def _(i):
        sl = pl.ds(i * L, L)
        acc = slab[sl]
        for t in range(1, L):
            acc = acc + slab[pl.ds(t * NBC + i * L, L)]
        fold[sl] = acc

    pltpu.sync_copy(fold, out_cnt.at[wid])

    pltpu.sync_copy(tasum.at[pl.ds(s * FSTRIPE, C)], lgb0)
    pltpu.sync_copy(tasum.at[pl.ds(s * FSTRIPE + C, C)], lgb1)

    @plsc.parallel_loop(0, CSTRIPE // L // 2, 1, unroll=2)
    def _(i):
        lo = i * 16 * L
        acc = jnp.zeros((L,), jnp.float32)
        for f in range(16):
            acc = acc + plsc.load_gather(lgb0, [lo + lanes * 16 + f])
        av0[pl.ds(i * L, L)] = acc

    @plsc.parallel_loop(0, CSTRIPE // L // 2, 1, unroll=2)
    def _(i):
        lo = i * 16 * L
        acc = jnp.zeros((L,), jnp.float32)
        for f in range(16):
            acc = acc + plsc.load_gather(lgb1, [lo + lanes * 16 + f])
        av0[pl.ds((CSTRIPE // 2) + i * L, L)] = acc

    pltpu.sync_copy(av0.at[pl.ds(0, CSTRIPE)],
                    out_asum.at[c, pl.ds(s * CSTRIPE, CSTRIPE)])


def _scan_body(cnt_hbm, asum_hbm, out_hbm,
               cbuf, nbuf, pbuf, am0b, am1b, ap0b, ap1b, stage, stagef, exv,
               exvf, outv, exch, exch2, semc, sema):
    c = lax.axis_index("c")
    s = lax.axis_index("s")
    b0 = s * STR

    for r in range(NW):
        pltpu.async_copy(cnt_hbm.at[r, pl.ds(b0, STR)],
                         cbuf.at[pl.ds(r * STR, STR)], semc)
    pltpu.async_copy(asum_hbm.at[0, pl.ds(b0, STR)], am0b, sema)
    pltpu.async_copy(asum_hbm.at[1, pl.ds(b0, STR)], am1b, sema)
    pltpu.async_copy(asum_hbm.at[0, pl.ds(NBC + b0, STR)], ap0b, sema)
    pltpu.async_copy(asum_hbm.at[1, pl.ds(NBC + b0, STR)], ap1b, sema)
    for r in range(NW):
        pltpu.make_async_copy(cnt_hbm.at[r, pl.ds(b0, STR)],
                              cbuf.at[pl.ds(r * STR, STR)], semc).wait()
    pltpu.make_async_copy(asum_hbm.at[0, pl.ds(b0, STR)], am0b, sema).wait()
    pltpu.make_async_copy(asum_hbm.at[1, pl.ds(b0, STR)], am1b, sema).wait()
    pltpu.make_async_copy(asum_hbm.at[0, pl.ds(NBC + b0, STR)], ap0b,
                          sema).wait()
    pltpu.make_async_copy(asum_hbm.at[1, pl.ds(NBC + b0, STR)], ap1b,
                          sema).wait()

    def merge_body(j, carry):
        sn, sp = carry
        sl = pl.ds(j * L, L)
        tot = jnp.zeros((L,), jnp.int32)
        pos = jnp.zeros((L,), jnp.int32)
        for r in range(NW):
            v = cbuf[pl.ds(r * STR + j * L, L)]
            tot = tot + (v & 0xFFFF)
            pos = pos + lax.shift_right_logical(v, 16)
        neg = tot - pos
        nbuf[sl] = neg
        pbuf[sl] = pos
        return sn + jnp.sum(neg), sp + jnp.sum(pos)
    sneg, spos = lax.fori_loop(0, STR // L, merge_body,
                               (jnp.int32(0), jnp.int32(0)))

    lanes = lax.broadcasted_iota(jnp.int32, (L,), 0)
    stage[...] = jnp.where(lanes == 0, sneg, jnp.where(lanes == 1, spos, 0))
    pltpu.sync_copy(stage, exch.at[pl.ds(s * L, L)])
    plsc.subcore_barrier()
    pltpu.sync_copy(exch, exv)
    negs_all = plsc.load_gather(exv, [lanes * L])
    poss_all = plsc.load_gather(exv, [lanes * L + 1])
    qbase = jnp.sum(jnp.where(lanes < s, negs_all, 0))
    rbase = jnp.sum(jnp.where(lanes < s, poss_all, 0))
    pf = jnp.sum(poss_all).astype(jnp.float32)

    def scan_body(j, carry):
        qc, rc, acc = carry
        sl = pl.ds(j * L, L)
        neg = nbuf[sl]
        pos = pbuf[sl]
        qv = plsc.cumsum(neg) - neg + qc
        rv = plsc.cumsum(pos) - pos + rc
        qf = qv.astype(jnp.float32)
        rf = rv.astype(jnp.float32)
        negf = neg.astype(jnp.float32)
        posf = pos.astype(jnp.float32)
        am = am0b[sl] + am1b[sl]
        ap = ap0b[sl] + ap1b[sl]
        gplus = 1.0 / jnp.maximum(pf + qf + 0.5 * negf, 0.25)
        u0 = pf + qf + 0.5 * (negf - 1.0)
        gminus = (pf - rf - 0.5 * posf) / jnp.maximum(u0 * (u0 + 1.0), 0.25)
        acc = acc + ap * gplus + am * gminus
        return qc + jnp.sum(neg), rc + jnp.sum(pos), acc

    _, _, acc = lax.fori_loop(0, STR // L, scan_body,
                              (qbase, rbase, jnp.zeros((L,), jnp.float32)))
    part = jnp.sum(acc)
    stagef[...] = jnp.where(lanes == 0, part, 0.0)
    pltpu.sync_copy(stagef, exch2.at[pl.ds(s * L, L)])
    plsc.subcore_barrier()

    @pl.when(jnp.logical_and(c == 0, s == 0))
    def _():
        pltpu.sync_copy(exch2, exvf)
        parts = plsc.load_gather(exvf, [lanes * L])
        total = jnp.sum(parts)
        outv[...] = jnp.full((L,), total, jnp.float32)
        pltpu.sync_copy(outv, out_hbm)


@functools.partial(jax.jit, static_argnames=())
def kernel(logits, targets):
    lg = logits.reshape(-1)
    tg = targets.reshape(-1)
    mesh = plsc.VectorSubcoreMesh(core_axis_name="c", subcore_axis_name="s")
    params = pltpu.CompilerParams(needs_layout_passes=False)

    hist = pl.kernel(
        _hist_body,
        out_type=(
            jax.ShapeDtypeStruct((NW, NBC), jnp.int32),
            jax.ShapeDtypeStruct((NC, 2 * NBC), jnp.float32),
        ),
        mesh=mesh,
        scratch_types=[
            pltpu.VMEM((C,), jnp.float32),
            pltpu.VMEM((C,), jnp.float32),
            pltpu.VMEM((C,), jnp.int32),
            pltpu.VMEM((C,), jnp.int32),
            pltpu.VMEM((C,), jnp.float32),
            pltpu.VMEM((C,), jnp.float32),
            pltpu.VMEM((C,), jnp.int32),
            pltpu.VMEM((C,), jnp.int32),
            pltpu.VMEM((NBC,), jnp.int32),
            pltpu.VMEM((SLAB,), jnp.int32),
            pltpu.VMEM_SHARED((2 * NBF,), jnp.float32),
            pltpu.SemaphoreType.DMA,
            pltpu.SemaphoreType.DMA,
            pltpu.SemaphoreType.DMA,
            pltpu.SemaphoreType.DMA,
            pltpu.SemaphoreType.DMA,
            pltpu.SemaphoreType.DMA,
        ],
        compiler_params=params,
    )
    cnt, asum = hist(lg, tg)

    scan = pl.kernel(
        _scan_body,
        out_type=jax.ShapeDtypeStruct((L,), jnp.float32),
        mesh=plsc.VectorSubcoreMesh(core_axis_name="c", subcore_axis_name="s"),
        scratch_types=[
            pltpu.VMEM((NW * STR,), jnp.int32),
            pltpu.VMEM((STR,), jnp.int32),
            pltpu.VMEM((STR,), jnp.int32),
            pltpu.VMEM((STR,), jnp.float32),
            pltpu.VMEM((STR,), jnp.float32),
            pltpu.VMEM((STR,), jnp.float32),
            pltpu.VMEM((STR,), jnp.float32),
            pltpu.VMEM((L,), jnp.int32),
            pltpu.VMEM((L,), jnp.float32),
            pltpu.VMEM((NS * L,), jnp.int32),
            pltpu.VMEM((NS * L,), jnp.float32),
            pltpu.VMEM((L,), jnp.float32),
            pltpu.VMEM_SHARED((NS * L,), jnp.int32),
            pltpu.VMEM_SHARED((NS * L,), jnp.float32),
            pltpu.SemaphoreType.DMA,
            pltpu.SemaphoreType.DMA,
        ],
        compiler_params=params,
    )
    out = scan(cnt, asum)
    return out[0]


# speedup vs baseline: 2.7219x; 1.3398x over previous
import functools

import jax
import jax.numpy as jnp
from jax import lax
from jax.experimental import pallas as pl
from jax.experimental.pallas import tpu as pltpu
from jax.experimental.pallas import tpu_sc as plsc

N = 8 * 512 * 512
NBF = 65536
NBC = 4096
NC, NS, L = 2, 16, 16
NW = NC * NS
PER_W = N // NW
C = 4096
CHUNKS = PER_W // C
VPC = C // L
FSTRIPE = 2 * NBF // NS
CSTRIPE = 2 * NBC // NS
SLAB = L * NBC
STR = NBC // NS


def _hist_body(lg_hbm, tg_hbm, out_cnt, out_asum,
               lgb0, lgb1, tgb0, tgb1, av0, av1, ai0, ai1, fold, slab, tasum,
               semlg0, semlg1, semtg0, semtg1, sems0, sems1):
    c = lax.axis_index("c")
    s = lax.axis_index("s")
    wid = c * NS + s
    img = wid >> 2
    rbase = (wid & 3) * 128
    lgb = (lgb0, lgb1)
    tgb = (tgb0, tgb1)
    av = (av0, av1)
    ai = (ai0, ai1)
    semlg = (semlg0, semlg1)
    semtg = (semtg0, semtg1)
    sems = (sems0, sems1)
    lanes = lax.broadcasted_iota(jnp.int32, (L,), 0)
    zi = jnp.zeros((L,), jnp.int32)
    zf = jnp.zeros((L,), jnp.float32)

    @plsc.parallel_loop(0, SLAB // L, 1, unroll=8)
    def _(j):
        slab[pl.ds(j * L, L)] = zi

    @plsc.parallel_loop(0, VPC, 1, unroll=8)
    def _(j):
        av0[pl.ds(j * L, L)] = zf

    pltpu.sync_copy(av0, tasum.at[pl.ds(s * FSTRIPE, C)])
    pltpu.sync_copy(av0, tasum.at[pl.ds(s * FSTRIPE + C, C)])
    plsc.subcore_barrier()

    pltpu.async_copy(lg_hbm.at[img, 0, pl.ds(rbase, 8), :], lgb0, semlg0)
    pltpu.async_copy(tg_hbm.at[img, 0, pl.ds(rbase, 8), :], tgb0, semtg0)

    def chunk_pair(g, _):
        for b in range(2):
            k = 2 * g + b
            nxt = k + 1

            @pl.when(nxt < CHUNKS)
            def _():
                nr = rbase + nxt * 8
                pltpu.async_copy(lg_hbm.at[img, 0, pl.ds(nr, 8), :],
                                 lgb[1 - b], semlg[1 - b])
                pltpu.async_copy(tg_hbm.at[img, 0, pl.ds(nr, 8), :],
                                 tgb[1 - b], semtg[1 - b])

            kr = rbase + k * 8
            pltpu.make_async_copy(lg_hbm.at[img, 0, pl.ds(kr, 8), :], lgb[b],
                                  semlg[b]).wait()
            pltpu.make_async_copy(tg_hbm.at[img, 0, pl.ds(kr, 8), :], tgb[b],
                                  semtg[b]).wait()

            @pl.when(k >= 2)
            def _():
                pltpu.make_async_copy(av[b], tasum.at[ai[b]], sems[b]).wait()

            lgbb, tgbb, avb, aib = lgb[b], tgb[b], av[b], ai[b]

            @plsc.parallel_loop(0, VPC, 1, unroll=4)
            def _(j):
                sl = pl.ds(j * L, L)
                rr = j >> 5
                cc = (j & 31) * L
                x = lgbb[rr, pl.ds(cc, L)]
                l = tgbb[rr, pl.ds(cc, L)]
                lf = l.astype(jnp.float32)
                e = 1.0 - x * (2.0 * lf - 1.0)
                a = jnp.maximum(e, 0.0)
                bu = lax.bitcast_convert_type(e, jnp.uint32)
                negm = lax.bitcast_convert_type(e, jnp.int32) < 0
                u = jnp.where(negm, ~bu, bu | jnp.uint32(0x80000000))
                inv = ~u
                b16 = (inv >> 16).astype(jnp.int32)
                b12 = (inv >> 20).astype(jnp.int32)
                plsc.addupdate_scatter(slab, [lanes * NBC + b12],
                                       1 + (l << 16))
                avb[sl] = a
                aib[sl] = b16 + (l << 16)

            pltpu.async_copy(av[b], tasum.at[ai[b]], sems[b], add=True)
        return 0
    lax.fori_loop(0, CHUNKS // 2, chunk_pair, 0)
    pltpu.make_async_copy(av0, tasum.at[ai0], sems0).wait()
    pltpu.make_async_copy(av1, tasum.at[ai1], sems1).wait()
    plsc.subcore_barrier()

    @plsc.parallel_loop(0, NBC // L, 1, unroll=2)
    def _(i):
        sl = pl.ds(i * L, L)
        acc = slab[sl]
        for t in range(1, L):
            acc = acc + slab[pl.ds(t * NBC + i * L, L)]
        fold[sl] = acc

    pltpu.sync_copy(fold, out_cnt.at[wid])

    pltpu.sync_copy(tasum.at[pl.ds(s * FSTRIPE, C)], av1)

    @plsc.parallel_loop(0, CSTRIPE // L // 2, 1, unroll=2)
    def _(i):
        lo = i * 16 * L
        acc = jnp.zeros((L,), jnp.float32)
        for f in range(16):
            acc = acc + plsc.load_gather(av1, [lo + lanes * 16 + f])
        av0[pl.ds(i * L, L)] = acc

    pltpu.sync_copy(tasum.at[pl.ds(s * FSTRIPE + C, C)], av1)

    @plsc.parallel_loop(0, CSTRIPE // L // 2, 1, unroll=2)
    def _(i):
        lo = i * 16 * L
        acc = jnp.zeros((L,), jnp.float32)
        for f in range(16):
            acc = acc + plsc.load_gather(av1, [lo + lanes * 16 + f])
        av0[pl.ds((CSTRIPE // 2) + i * L, L)] = acc

    pltpu.sync_copy(av0.at[pl.ds(0, CSTRIPE)],
                    out_asum.at[c, pl.ds(s * CSTRIPE, CSTRIPE)])


def _scan_body(cnt_hbm, asum_hbm, out_hbm,
               cbuf, nbuf, pbuf, am0b, am1b, ap0b, ap1b, stage, stagef, exv,
               exvf, outv, exch, exch2, semc, sema):
    c = lax.axis_index("c")
    s = lax.axis_index("s")
    b0 = s * STR

    for r in range(NW):
        pltpu.async_copy(cnt_hbm.at[r, pl.ds(b0, STR)],
                         cbuf.at[pl.ds(r * STR, STR)], semc)
    pltpu.async_copy(asum_hbm.at[0, pl.ds(b0, STR)], am0b, sema)
    pltpu.async_copy(asum_hbm.at[1, pl.ds(b0, STR)], am1b, sema)
    pltpu.async_copy(asum_hbm.at[0, pl.ds(NBC + b0, STR)], ap0b, sema)
    pltpu.async_copy(asum_hbm.at[1, pl.ds(NBC + b0, STR)], ap1b, sema)
    for r in range(NW):
        pltpu.make_async_copy(cnt_hbm.at[r, pl.ds(b0, STR)],
                              cbuf.at[pl.ds(r * STR, STR)], semc).wait()
    pltpu.make_async_copy(asum_hbm.at[0, pl.ds(b0, STR)], am0b, sema).wait()
    pltpu.make_async_copy(asum_hbm.at[1, pl.ds(b0, STR)], am1b, sema).wait()
    pltpu.make_async_copy(asum_hbm.at[0, pl.ds(NBC + b0, STR)], ap0b,
                          sema).wait()
    pltpu.make_async_copy(asum_hbm.at[1, pl.ds(NBC + b0, STR)], ap1b,
                          sema).wait()

    def merge_body(j, carry):
        sn, sp = carry
        sl = pl.ds(j * L, L)
        tot = jnp.zeros((L,), jnp.int32)
        pos = jnp.zeros((L,), jnp.int32)
        for r in range(NW):
            v = cbuf[pl.ds(r * STR + j * L, L)]
            tot = tot + (v & 0xFFFF)
            pos = pos + lax.shift_right_logical(v, 16)
        neg = tot - pos
        nbuf[sl] = neg
        pbuf[sl] = pos
        return sn + jnp.sum(neg), sp + jnp.sum(pos)
    sneg, spos = lax.fori_loop(0, STR // L, merge_body,
                               (jnp.int32(0), jnp.int32(0)))

    lanes = lax.broadcasted_iota(jnp.int32, (L,), 0)
    stage[...] = jnp.where(lanes == 0, sneg, jnp.where(lanes == 1, spos, 0))
    pltpu.sync_copy(stage, exch.at[pl.ds(s * L, L)])
    plsc.subcore_barrier()
    pltpu.sync_copy(exch, exv)
    negs_all = plsc.load_gather(exv, [lanes * L])
    poss_all = plsc.load_gather(exv, [lanes * L + 1])
    qbase = jnp.sum(jnp.where(lanes < s, negs_all, 0))
    rbase = jnp.sum(jnp.where(lanes < s, poss_all, 0))
    pf = jnp.sum(poss_all).astype(jnp.float32)

    def scan_body(j, carry):
        qc, rc, acc = carry
        sl = pl.ds(j * L, L)
        neg = nbuf[sl]
        pos = pbuf[sl]
        qv = plsc.cumsum(neg) - neg + qc
        rv = plsc.cumsum(pos) - pos + rc
        qf = qv.astype(jnp.float32)
        rf = rv.astype(jnp.float32)
        negf = neg.astype(jnp.float32)
        posf = pos.astype(jnp.float32)
        am = am0b[sl] + am1b[sl]
        ap = ap0b[sl] + ap1b[sl]
        gplus = 1.0 / jnp.maximum(pf + qf + 0.5 * negf, 0.25)
        u0 = pf + qf + 0.5 * (negf - 1.0)
        gminus = (pf - rf - 0.5 * posf) / jnp.maximum(u0 * (u0 + 1.0), 0.25)
        acc = acc + ap * gplus + am * gminus
        return qc + jnp.sum(neg), rc + jnp.sum(pos), acc

    _, _, acc = lax.fori_loop(0, STR // L, scan_body,
                              (qbase, rbase, jnp.zeros((L,), jnp.float32)))
    part = jnp.sum(acc)
    stagef[...] = jnp.where(lanes == 0, part, 0.0)
    pltpu.sync_copy(stagef, exch2.at[pl.ds(s * L, L)])
    plsc.subcore_barrier()

    @pl.when(jnp.logical_and(c == 0, s == 0))
    def _():
        pltpu.sync_copy(exch2, exvf)
        parts = plsc.load_gather(exvf, [lanes * L])
        total = jnp.sum(parts)
        outv[...] = jnp.full((L,), total, jnp.float32)
        pltpu.sync_copy(outv, out_hbm)


@functools.partial(jax.jit, static_argnames=())
def kernel(logits, targets):
    lg = logits
    tg = targets
    mesh = plsc.VectorSubcoreMesh(core_axis_name="c", subcore_axis_name="s")
    params = pltpu.CompilerParams(needs_layout_passes=False,
                                  use_tc_tiling_on_sc=True)

    hist = pl.kernel(
        _hist_body,
        out_type=(
            jax.ShapeDtypeStruct((NW, NBC), jnp.int32),
            jax.ShapeDtypeStruct((NC, 2 * NBC), jnp.float32),
        ),
        mesh=mesh,
        scratch_types=[
            pltpu.VMEM((8, 512), jnp.float32),
            pltpu.VMEM((8, 512), jnp.float32),
            pltpu.VMEM((8, 512), jnp.int32),
            pltpu.VMEM((8, 512), jnp.int32),
            pltpu.VMEM((C,), jnp.float32),
            pltpu.VMEM((C,), jnp.float32),
            pltpu.VMEM((C,), jnp.int32),
            pltpu.VMEM((C,), jnp.int32),
            pltpu.VMEM((NBC,), jnp.int32),
            pltpu.VMEM((SLAB,), jnp.int32),
            pltpu.VMEM_SHARED((2 * NBF,), jnp.float32),
            pltpu.SemaphoreType.DMA,
            pltpu.SemaphoreType.DMA,
            pltpu.SemaphoreType.DMA,
            pltpu.SemaphoreType.DMA,
            pltpu.SemaphoreType.DMA,
            pltpu.SemaphoreType.DMA,
        ],
        compiler_params=params,
    )
    cnt, asum = hist(lg, tg)

    scan = pl.kernel(
        _scan_body,
        out_type=jax.ShapeDtypeStruct((L,), jnp.float32),
        mesh=plsc.VectorSubcoreMesh(core_axis_name="c", subcore_axis_name="s"),
        scratch_types=[
            pltpu.VMEM((NW * STR,), jnp.int32),
            pltpu.VMEM((STR,), jnp.int32),
            pltpu.VMEM((STR,), jnp.int32),
            pltpu.VMEM((STR,), jnp.float32),
            pltpu.VMEM((STR,), jnp.float32),
            pltpu.VMEM((STR,), jnp.float32),
            pltpu.VMEM((STR,), jnp.float32),
            pltpu.VMEM((L,), jnp.int32),
            pltpu.VMEM((L,), jnp.float32),
            pltpu.VMEM((NS * L,), jnp.int32),
            pltpu.VMEM((NS * L,), jnp.float32),
            pltpu.VMEM((L,), jnp.float32),
            pltpu.VMEM_SHARED((NS * L,), jnp.int32),
            pltpu.VMEM_SHARED((NS * L,), jnp.float32),
            pltpu.SemaphoreType.DMA,
            pltpu.SemaphoreType.DMA,
        ],
        compiler_params=params,
    )
    out = scan(cnt, asum)
    return out[0]
